# Initial kernel scaffold; baseline (speedup 1.0000x reference)
import jax
import jax.numpy as jnp
from jax.experimental import pallas as pl


def kernel(x, edge_index, node_types, Wq, Wk, Wv, Wo, bo, g1, b1, Wt, bt, Wm1, bm1, Wm2, bm2, Wa1, ba1, Wa2, ba2, Wop, bop, g2, b2):
    raise NotImplementedError("write your pallas kernel here")


# speedup vs baseline: 18.5905x; 18.5905x over previous
import functools
import jax
import jax.numpy as jnp
from jax import lax
from jax.experimental import pallas as pl
from jax.experimental.pallas import tpu as pltpu
from jax.experimental.pallas import tpu_sc as plsc

N = 10000
E = 320000
D = 128
H = 8
HD = 16
T = 6
ALPHA = 0.2

NC = 2
NS = 16
NW = NC * NS
EPW = E // NW
CHUNK = 40
NCHUNK = EPW // CHUNK
N_PAD = 10112
RPT = N_PAD // NS

BLK = 1000
GRID = N // BLK


def _dotT(a, w):
    return lax.dot_general(a, w, (((1,), (1,)), ((), ())),
                           preferred_element_type=jnp.float32)


def _ln_blk(v, g, b, eps=1e-5):
    mu = jnp.mean(v, axis=-1, keepdims=True)
    var = jnp.mean((v - mu) ** 2, axis=-1, keepdims=True)
    return (v - mu) / jnp.sqrt(var + eps) * g + b


def _pre_body(x_ref, nt_ref, wq_ref, wk_ref, wv_ref, wt_ref, bt_ref,
              w1a_ref, w1b_ref, q_o, k_o, v_o, h_o, u_o, vg_o):
    x = x_ref[...]
    q_o[...] = _dotT(x, wq_ref[...])
    k_o[...] = _dotT(x, wk_ref[...])
    v_o[...] = _dotT(x, wv_ref[...])
    nt = nt_ref[...]
    h = jnp.zeros((BLK, D), jnp.float32)
    for t in range(T):
        ht = _dotT(x, wt_ref[t]) + bt_ref[t]
        h = h + jnp.where(nt == t, ht, 0.0)
    h_o[...] = h
    u_o[...] = _dotT(h, w1a_ref[...])
    vg_o[...] = _dotT(h, w1b_ref[...])


def _run_pre(x, nt2, Wq, Wk, Wv, Wt, bt, W1a, W1b):
    blk = lambda i: (i, 0)
    full = lambda i: (0, 0)
    full3 = lambda i: (0, 0, 0)
    o = jax.ShapeDtypeStruct((N, D), jnp.float32)
    return pl.pallas_call(
        _pre_body,
        grid=(GRID,),
        in_specs=[
            pl.BlockSpec((BLK, D), blk),
            pl.BlockSpec((BLK, 1), blk),
            pl.BlockSpec((D, D), full),
            pl.BlockSpec((D, D), full),
            pl.BlockSpec((D, D), full),
            pl.BlockSpec((T, D, D), full3),
            pl.BlockSpec((T, D), full),
            pl.BlockSpec((D, D), full),
            pl.BlockSpec((D, D), full),
        ],
        out_specs=[pl.BlockSpec((BLK, D), blk)] * 6,
        out_shape=[o] * 6,
    )(x, nt2, Wq, Wk, Wv, Wt, bt, W1a, W1b)


def _edges_sc(eir, eic, q, k, v, u, vg, bm1, aggp, asump, degp,
              idxr, idxc, b0, b1, b2, msg, bm1v, acc):
    cid = lax.axis_index("c")
    sid = lax.axis_index("s")
    wid = sid * NC + cid
    ebase = wid * EPW

    pltpu.sync_copy(bm1, bm1v)

    iot = lax.iota(jnp.int32, 16)

    def zmsg(r, _):
        for hh in range(8):
            msg[r, pl.ds(hh * 16, 16)] = jnp.zeros((16,), jnp.float32)
        return 0

    def zero_acc():
        lax.fori_loop(0, CHUNK, zmsg, 0)
        for j in range(RPT // CHUNK):
            pltpu.sync_copy(msg, acc.at[pl.ds(sid * RPT + j * CHUNK, CHUNK)])
        rem = RPT % CHUNK
        if rem:
            pltpu.sync_copy(msg.at[pl.ds(0, rem)],
                            acc.at[pl.ds(sid * RPT + RPT - rem, rem)])

    def read_out(dst, buf):
        for j in range(RPT // CHUNK):
            r0 = sid * RPT + j * CHUNK
            pltpu.sync_copy(acc.at[pl.ds(r0, CHUNK)], buf)
            pltpu.sync_copy(buf, dst.at[cid, pl.ds(r0, CHUNK)])
        rem = RPT % CHUNK
        if rem:
            r0 = sid * RPT + RPT - rem
            pltpu.sync_copy(acc.at[pl.ds(r0, rem)], buf.at[pl.ds(0, rem)])
            pltpu.sync_copy(buf.at[pl.ds(0, rem)],
                            dst.at[cid, pl.ds(r0, rem)])

    zero_acc()
    plsc.subcore_barrier()

    bm1_h = [bm1v[pl.ds(hh * HD, HD)] for hh in range(H)]

    def gcn_edge(e, _):
        for hh in range(H):
            uh = b0[e, pl.ds(hh * HD, HD)]
            vh = b1[e, pl.ds(hh * HD, HD)]
            msg[e, pl.ds(hh * HD, HD)] = jnp.maximum(uh + vh + bm1_h[hh], 0.0)
        return 0

    def gcn_chunk(i, _):
        base = ebase + i * CHUNK
        pltpu.sync_copy(eir.at[pl.ds(base, CHUNK)], idxr)
        pltpu.sync_copy(eic.at[pl.ds(base, CHUNK)], idxc)
        pltpu.sync_copy(u.at[idxr], b0)
        pltpu.sync_copy(vg.at[idxc], b1)
        lax.fori_loop(0, CHUNK, gcn_edge, 0)
        pltpu.sync_copy(msg, acc.at[idxc], add=True)
        return 0

    lax.fori_loop(0, NCHUNK, gcn_chunk, 0)
    plsc.subcore_barrier()

    read_out(asump, msg)
    zero_acc()
    plsc.subcore_barrier()

    def omsg(r, _):
        for hh in range(8):
            msg[r, pl.ds(hh * 16, 16)] = jnp.full((16,), 1.0, jnp.float32)
        return 0

    def deg_chunk(i, _):
        base = ebase + i * CHUNK
        pltpu.sync_copy(eic.at[pl.ds(base, CHUNK)], idxc)
        pltpu.sync_copy(msg, acc.at[idxc], add=True)
        return 0

    lax.fori_loop(0, CHUNK, omsg, 0)
    lax.fori_loop(0, NCHUNK, deg_chunk, 0)
    plsc.subcore_barrier()
    read_out(degp, msg)
    zero_acc()
    plsc.subcore_barrier()

    perms = [iot ^ (1 << j) for j in range(4)]
    gdn = lax.GatherDimensionNumbers(offset_dims=(), collapsed_slice_dims=(0,),
                                     start_index_map=(0,))

    def vsum(xv):
        for p in perms:
            xv = xv + lax.gather(xv, p[:, None], gdn, (1,),
                                 mode=lax.GatherScatterMode.PROMISE_IN_BOUNDS)
        return xv

    def gat_edge(e, _):
        svals = []
        for hh in range(H):
            qh = b0[e, pl.ds(hh * HD, HD)]
            kh = b1[e, pl.ds(hh * HD, HD)]
            s = vsum(qh * kh) * 0.25
            s = jnp.where(s > 0.0, s, ALPHA * s)
            svals.append(s)
        m = svals[0]
        for hh in range(1, H):
            m = jnp.maximum(m, svals[hh])
        evs = [jnp.exp(sv - m) for sv in svals]
        tot = evs[0]
        for hh in range(1, H):
            tot = tot + evs[hh]
        inv = 1.0 / tot
        for hh in range(H):
            vh = b2[e, pl.ds(hh * HD, HD)]
            msg[e, pl.ds(hh * HD, HD)] = evs[hh] * inv * vh
        return 0

    def gat_chunk(i, _):
        base = ebase + i * CHUNK
        pltpu.sync_copy(eir.at[pl.ds(base, CHUNK)], idxr)
        pltpu.sync_copy(eic.at[pl.ds(base, CHUNK)], idxc)
        pltpu.sync_copy(q.at[idxr], b0)
        pltpu.sync_copy(k.at[idxc], b1)
        pltpu.sync_copy(v.at[idxr], b2)
        lax.fori_loop(0, CHUNK, gat_edge, 0)
        pltpu.sync_copy(msg, acc.at[idxc], add=True)
        return 0

    lax.fori_loop(0, NCHUNK, gat_chunk, 0)
    plsc.subcore_barrier()
    read_out(aggp, msg)


_run_edges = functools.partial(
    pl.kernel,
    out_type=[jax.ShapeDtypeStruct((NC, N_PAD, D), jnp.float32),
              jax.ShapeDtypeStruct((NC, N_PAD, D), jnp.float32),
              jax.ShapeDtypeStruct((NC, N_PAD, D), jnp.float32)],
    mesh=plsc.VectorSubcoreMesh(core_axis_name="c", subcore_axis_name="s"),
    scratch_types=[
        pltpu.VMEM((CHUNK,), jnp.int32),
        pltpu.VMEM((CHUNK,), jnp.int32),
        pltpu.VMEM((CHUNK, D), jnp.float32),
        pltpu.VMEM((CHUNK, D), jnp.float32),
        pltpu.VMEM((CHUNK, D), jnp.float32),
        pltpu.VMEM((CHUNK, D), jnp.float32),
        pltpu.VMEM((D,), jnp.float32),
        pltpu.VMEM_SHARED((N_PAD, D), jnp.float32),
    ],
)(_edges_sc)


def _post_body(x_ref, h_ref, a0_ref, a1_ref, s0_ref, s1_ref, d0_ref, d1_ref,
               wo_ref, bo_ref, g1_ref, b1_ref, wm2_ref, bm2_ref,
               wa1_ref, ba1_ref, wa2_ref, ba2_ref, wop_ref, bop_ref,
               g2_ref, b2_ref, out_ref):
    x = x_ref[...]
    h = h_ref[...]
    agg = a0_ref[...] + a1_ref[...]
    gat = _ln_blk(_dotT(agg, wo_ref[...]) + bo_ref[...] + x,
                  g1_ref[...], b1_ref[...])
    asum = s0_ref[...] + s1_ref[...]
    deg = (d0_ref[...] + d1_ref[...])[:, 0:1]
    a = _dotT(asum, wm2_ref[...]) + deg * bm2_ref[...]
    a = _dotT(jnp.maximum(_dotT(a, wa1_ref[...]) + ba1_ref[...], 0.0),
              wa2_ref[...]) + ba2_ref[...]
    hout = _dotT(h + a, wop_ref[...]) + bop_ref[...]
    gcn = _ln_blk(hout + x, g2_ref[...], b2_ref[...])
    out_ref[...] = 0.5 * (gat + gcn)


def _run_post(x, h, a0, a1, s0, s1, d0, d1, Wo, bo2, g12, b12, Wm2, bm22,
              Wa1, ba12, Wa2, ba22, Wop, bop2, g22, b22):
    blk = lambda i: (i, 0)
    full = lambda i: (0, 0)
    nblk = pl.BlockSpec((BLK, D), blk)
    wfull = pl.BlockSpec((D, D), full)
    bfull = pl.BlockSpec((1, D), full)
    return pl.pallas_call(
        _post_body,
        grid=(GRID,),
        in_specs=[nblk, nblk, nblk, nblk, nblk, nblk, nblk, nblk,
                  wfull, bfull, bfull, bfull, wfull, bfull,
                  wfull, bfull, wfull, bfull, wfull, bfull,
                  bfull, bfull],
        out_specs=nblk,
        out_shape=jax.ShapeDtypeStruct((N, D), jnp.float32),
    )(x, h, a0, a1, s0, s1, d0, d1, Wo, bo2, g12, b12, Wm2, bm22,
      Wa1, ba12, Wa2, ba22, Wop, bop2, g22, b22)


def kernel(x, edge_index, node_types, Wq, Wk, Wv, Wo, bo, g1, b1, Wt, bt,
           Wm1, bm1, Wm2, bm2, Wa1, ba1, Wa2, ba2, Wop, bop, g2, b2):
    W1a = Wm1[:, :D]
    W1b = Wm1[:, D:]
    nt2 = node_types.reshape(N, 1)
    r2 = lambda p: p.reshape(1, D)

    eir = edge_index[0]
    eic = edge_index[1]
    q, k, v, h, u, vg = _run_pre(x, nt2, Wq, Wk, Wv, Wt, bt, W1a, W1b)
    aggp, asump, degp = _run_edges(eir, eic, q, k, v, u, vg, bm1)
    aggp = aggp[:, :N]
    asump = asump[:, :N]
    degp = degp[:, :N]

    return _run_post(x, h, aggp[0], aggp[1], asump[0], asump[1],
                     degp[0], degp[1],
                     Wo, r2(bo), r2(g1), r2(b1),
                     Wm2, r2(bm2), Wa1, r2(ba1), Wa2, r2(ba2),
                     Wop, r2(bop), r2(g2), r2(b2))


# speedup vs baseline: 26.0132x; 1.3993x over previous
import functools
import jax
import jax.numpy as jnp
from jax import lax
from jax.experimental import pallas as pl
from jax.experimental.pallas import tpu as pltpu
from jax.experimental.pallas import tpu_sc as plsc

N = 10000
E = 320000
D = 128
H = 8
HD = 16
T = 6
ALPHA = 0.2

NC = 2
NS = 16
NW = NC * NS
EPW = E // NW
CHUNK = 40
NCHUNK = EPW // CHUNK
N_PAD = 10112
RPT = N_PAD // NS

BLK = 1000
GRID = N // BLK


def _dotT(a, w):
    return lax.dot_general(a, w, (((1,), (1,)), ((), ())),
                           preferred_element_type=jnp.float32)


def _ln_blk(v, g, b, eps=1e-5):
    mu = jnp.mean(v, axis=-1, keepdims=True)
    var = jnp.mean((v - mu) ** 2, axis=-1, keepdims=True)
    return (v - mu) / jnp.sqrt(var + eps) * g + b


def _pre_body(x_ref, nt_ref, wq_ref, wk_ref, wv_ref, wt_ref, bt_ref,
              w1a_ref, w1b_ref, q_o, k_o, v_o, h_o, u_o, vg_o):
    x = x_ref[...]
    q_o[...] = _dotT(x, wq_ref[...])
    k_o[...] = _dotT(x, wk_ref[...])
    v_o[...] = _dotT(x, wv_ref[...])
    nt = nt_ref[...]
    h = jnp.zeros((BLK, D), jnp.float32)
    for t in range(T):
        ht = _dotT(x, wt_ref[t]) + bt_ref[t]
        h = h + jnp.where(nt == t, ht, 0.0)
    h_o[...] = h
    u_o[...] = _dotT(h, w1a_ref[...])
    vg_o[...] = _dotT(h, w1b_ref[...])


def _run_pre(x, nt2, Wq, Wk, Wv, Wt, bt, W1a, W1b):
    blk = lambda i: (i, 0)
    full = lambda i: (0, 0)
    full3 = lambda i: (0, 0, 0)
    o = jax.ShapeDtypeStruct((N, D), jnp.float32)
    return pl.pallas_call(
        _pre_body,
        grid=(GRID,),
        in_specs=[
            pl.BlockSpec((BLK, D), blk),
            pl.BlockSpec((BLK, 1), blk),
            pl.BlockSpec((D, D), full),
            pl.BlockSpec((D, D), full),
            pl.BlockSpec((D, D), full),
            pl.BlockSpec((T, D, D), full3),
            pl.BlockSpec((T, D), full),
            pl.BlockSpec((D, D), full),
            pl.BlockSpec((D, D), full),
        ],
        out_specs=[pl.BlockSpec((BLK, D), blk)] * 6,
        out_shape=[o] * 6,
    )(x, nt2, Wq, Wk, Wv, Wt, bt, W1a, W1b)


def _edges_sc(eir, eic, q, k, v, u, vg, bm1, aggp, asump, degp,
              idxr, idxc, b0, b1, b2, msg, bm1v, acc, sem):
    cid = lax.axis_index("c")
    sid = lax.axis_index("s")
    wid = sid * NC + cid
    ebase = wid * EPW

    pltpu.sync_copy(bm1, bm1v)

    iot = lax.iota(jnp.int32, 16)

    def zmsg(r, _):
        for hh in range(8):
            msg[r, pl.ds(hh * 16, 16)] = jnp.zeros((16,), jnp.float32)
        return 0

    def zero_acc():
        lax.fori_loop(0, CHUNK, zmsg, 0)
        for j in range(RPT // CHUNK):
            pltpu.sync_copy(msg, acc.at[pl.ds(sid * RPT + j * CHUNK, CHUNK)])
        rem = RPT % CHUNK
        if rem:
            pltpu.sync_copy(msg.at[pl.ds(0, rem)],
                            acc.at[pl.ds(sid * RPT + RPT - rem, rem)])

    def read_out(dst, buf):
        for j in range(RPT // CHUNK):
            r0 = sid * RPT + j * CHUNK
            pltpu.sync_copy(acc.at[pl.ds(r0, CHUNK)], buf)
            pltpu.sync_copy(buf, dst.at[cid, pl.ds(r0, CHUNK)])
        rem = RPT % CHUNK
        if rem:
            r0 = sid * RPT + RPT - rem
            pltpu.sync_copy(acc.at[pl.ds(r0, rem)], buf.at[pl.ds(0, rem)])
            pltpu.sync_copy(buf.at[pl.ds(0, rem)],
                            dst.at[cid, pl.ds(r0, rem)])

    zero_acc()
    plsc.subcore_barrier()

    bm1_h = [bm1v[pl.ds(hh * HD, HD)] for hh in range(H)]

    def gcn_edge(e, _):
        for hh in range(H):
            uh = b0[e, pl.ds(hh * HD, HD)]
            vh = b1[e, pl.ds(hh * HD, HD)]
            msg[e, pl.ds(hh * HD, HD)] = jnp.maximum(uh + vh + bm1_h[hh], 0.0)
        return 0

    def gcn_chunk(i, _):
        base = ebase + i * CHUNK
        c0 = pltpu.async_copy(eir.at[pl.ds(base, CHUNK)], idxr, sem)
        c1 = pltpu.async_copy(eic.at[pl.ds(base, CHUNK)], idxc, sem)
        c0.wait()
        c1.wait()
        g0 = pltpu.async_copy(u.at[idxr], b0, sem)
        g1 = pltpu.async_copy(vg.at[idxc], b1, sem)
        g0.wait()
        g1.wait()
        lax.fori_loop(0, CHUNK, gcn_edge, 0)
        pltpu.sync_copy(msg, acc.at[idxc], add=True)
        return 0

    lax.fori_loop(0, NCHUNK, gcn_chunk, 0)
    plsc.subcore_barrier()

    read_out(asump, msg)
    zero_acc()
    plsc.subcore_barrier()

    def omsg(r, _):
        for hh in range(8):
            msg[r, pl.ds(hh * 16, 16)] = jnp.full((16,), 1.0, jnp.float32)
        return 0

    def deg_chunk(i, _):
        base = ebase + i * CHUNK
        pltpu.sync_copy(eic.at[pl.ds(base, CHUNK)], idxc)
        pltpu.sync_copy(msg, acc.at[idxc], add=True)
        return 0

    lax.fori_loop(0, CHUNK, omsg, 0)
    lax.fori_loop(0, NCHUNK, deg_chunk, 0)
    plsc.subcore_barrier()
    read_out(degp, msg)
    zero_acc()
    plsc.subcore_barrier()

    perms = [iot ^ (1 << j) for j in range(4)]
    gdn = lax.GatherDimensionNumbers(offset_dims=(), collapsed_slice_dims=(0,),
                                     start_index_map=(0,))

    def vsum(xv):
        for p in perms:
            xv = xv + lax.gather(xv, p[:, None], gdn, (1,),
                                 mode=lax.GatherScatterMode.PROMISE_IN_BOUNDS)
        return xv

    def gat_edge(e, _):
        svals = []
        for hh in range(H):
            qh = b0[e, pl.ds(hh * HD, HD)]
            kh = b1[e, pl.ds(hh * HD, HD)]
            s = vsum(qh * kh) * 0.25
            s = jnp.where(s > 0.0, s, ALPHA * s)
            svals.append(s)
        m = svals[0]
        for hh in range(1, H):
            m = jnp.maximum(m, svals[hh])
        evs = [jnp.exp(sv - m) for sv in svals]
        tot = evs[0]
        for hh in range(1, H):
            tot = tot + evs[hh]
        inv = 1.0 / tot
        for hh in range(H):
            vh = b2[e, pl.ds(hh * HD, HD)]
            msg[e, pl.ds(hh * HD, HD)] = evs[hh] * inv * vh
        return 0

    def gat_chunk(i, _):
        base = ebase + i * CHUNK
        c0 = pltpu.async_copy(eir.at[pl.ds(base, CHUNK)], idxr, sem)
        c1 = pltpu.async_copy(eic.at[pl.ds(base, CHUNK)], idxc, sem)
        c0.wait()
        c1.wait()
        g0 = pltpu.async_copy(q.at[idxr], b0, sem)
        g1 = pltpu.async_copy(k.at[idxc], b1, sem)
        g2 = pltpu.async_copy(v.at[idxr], b2, sem)
        g0.wait()
        g1.wait()
        g2.wait()
        lax.fori_loop(0, CHUNK, gat_edge, 0)
        pltpu.sync_copy(msg, acc.at[idxc], add=True)
        return 0

    lax.fori_loop(0, NCHUNK, gat_chunk, 0)
    plsc.subcore_barrier()
    read_out(aggp, msg)


_run_edges = functools.partial(
    pl.kernel,
    out_type=[jax.ShapeDtypeStruct((NC, N_PAD, D), jnp.float32),
              jax.ShapeDtypeStruct((NC, N_PAD, D), jnp.float32),
              jax.ShapeDtypeStruct((NC, N_PAD, D), jnp.float32)],
    mesh=plsc.VectorSubcoreMesh(core_axis_name="c", subcore_axis_name="s"),
    scratch_types=[
        pltpu.VMEM((CHUNK,), jnp.int32),
        pltpu.VMEM((CHUNK,), jnp.int32),
        pltpu.VMEM((CHUNK, D), jnp.float32),
        pltpu.VMEM((CHUNK, D), jnp.float32),
        pltpu.VMEM((CHUNK, D), jnp.float32),
        pltpu.VMEM((CHUNK, D), jnp.float32),
        pltpu.VMEM((D,), jnp.float32),
        pltpu.VMEM_SHARED((N_PAD, D), jnp.float32),
        pltpu.SemaphoreType.DMA,
    ],
)(_edges_sc)


def _post_body(x_ref, h_ref, a0_ref, a1_ref, s0_ref, s1_ref, d0_ref, d1_ref,
               wo_ref, bo_ref, g1_ref, b1_ref, wm2_ref, bm2_ref,
               wa1_ref, ba1_ref, wa2_ref, ba2_ref, wop_ref, bop_ref,
               g2_ref, b2_ref, out_ref):
    x = x_ref[...]
    h = h_ref[...]
    agg = a0_ref[...] + a1_ref[...]
    gat = _ln_blk(_dotT(agg, wo_ref[...]) + bo_ref[...] + x,
                  g1_ref[...], b1_ref[...])
    asum = s0_ref[...] + s1_ref[...]
    deg = (d0_ref[...] + d1_ref[...])[:, 0:1]
    a = _dotT(asum, wm2_ref[...]) + deg * bm2_ref[...]
    a = _dotT(jnp.maximum(_dotT(a, wa1_ref[...]) + ba1_ref[...], 0.0),
              wa2_ref[...]) + ba2_ref[...]
    hout = _dotT(h + a, wop_ref[...]) + bop_ref[...]
    gcn = _ln_blk(hout + x, g2_ref[...], b2_ref[...])
    out_ref[...] = 0.5 * (gat + gcn)


def _run_post(x, h, a0, a1, s0, s1, d0, d1, Wo, bo2, g12, b12, Wm2, bm22,
              Wa1, ba12, Wa2, ba22, Wop, bop2, g22, b22):
    blk = lambda i: (i, 0)
    full = lambda i: (0, 0)
    nblk = pl.BlockSpec((BLK, D), blk)
    wfull = pl.BlockSpec((D, D), full)
    bfull = pl.BlockSpec((1, D), full)
    return pl.pallas_call(
        _post_body,
        grid=(GRID,),
        in_specs=[nblk, nblk, nblk, nblk, nblk, nblk, nblk, nblk,
                  wfull, bfull, bfull, bfull, wfull, bfull,
                  wfull, bfull, wfull, bfull, wfull, bfull,
                  bfull, bfull],
        out_specs=nblk,
        out_shape=jax.ShapeDtypeStruct((N, D), jnp.float32),
    )(x, h, a0, a1, s0, s1, d0, d1, Wo, bo2, g12, b12, Wm2, bm22,
      Wa1, ba12, Wa2, ba22, Wop, bop2, g22, b22)


def kernel(x, edge_index, node_types, Wq, Wk, Wv, Wo, bo, g1, b1, Wt, bt,
           Wm1, bm1, Wm2, bm2, Wa1, ba1, Wa2, ba2, Wop, bop, g2, b2):
    W1a = Wm1[:, :D]
    W1b = Wm1[:, D:]
    nt2 = node_types.reshape(N, 1)
    r2 = lambda p: p.reshape(1, D)

    eir = edge_index[0]
    eic = edge_index[1]
    q, k, v, h, u, vg = _run_pre(x, nt2, Wq, Wk, Wv, Wt, bt, W1a, W1b)
    aggp, asump, degp = _run_edges(eir, eic, q, k, v, u, vg, bm1)
    aggp = aggp[:, :N]
    asump = asump[:, :N]
    degp = degp[:, :N]

    return _run_post(x, h, aggp[0], aggp[1], asump[0], asump[1],
                     degp[0], degp[1],
                     Wo, r2(bo), r2(g1), r2(b1),
                     Wm2, r2(bm2), Wa1, r2(ba1), Wa2, r2(ba2),
                     Wop, r2(bop), r2(g2), r2(b2))
